# Initial kernel scaffold; baseline (speedup 1.0000x reference)
import jax
import jax.numpy as jnp
from jax.experimental import pallas as pl


def kernel(x, w, gamma, beta):
    raise NotImplementedError("write your pallas kernel here")


# speedup vs baseline: 1.4348x; 1.4348x over previous
from functools import partial

import jax
import jax.numpy as jnp
from jax.experimental import pallas as pl
from jax.experimental.pallas import tpu as pltpu

_C = 3
_K = 3
_EPS = 1e-5
_VMEM_BYTES = 64 * 1024 * 1024


def _conv_stats_kernel(w_ref, x_ref, y_ref, stats_ref, sacc_ref, qacc_ref):
    step = pl.program_id(1)
    inner = pl.num_programs(1)
    tile_n, _, length = x_ref.shape

    col = jax.lax.broadcasted_iota(jnp.int32, (tile_n, length), 1)
    first = col == 0
    last = col == length - 1

    accs = [None, None, None]
    for ci in range(_C):
        xc = x_ref[:, ci, :]
        xm = jnp.where(first, 0.0, pltpu.roll(xc, shift=1, axis=1))
        xp = jnp.where(last, 0.0, pltpu.roll(xc, shift=length - 1, axis=1))
        for co in range(_C):
            base = (co * _C + ci) * _K
            t = w_ref[base] * xm + w_ref[base + 1] * xc + w_ref[base + 2] * xp
            accs[co] = t if ci == 0 else accs[co] + t

    @pl.when(step == 0)
    def _init():
        sacc_ref[...] = jnp.zeros_like(sacc_ref)
        qacc_ref[...] = jnp.zeros_like(qacc_ref)

    for co in range(_C):
        y = accs[co]
        y_ref[co] = y.astype(y_ref.dtype)
        sacc_ref[co] = sacc_ref[co] + y
        qacc_ref[co] = qacc_ref[co] + y * y

    @pl.when(step == inner - 1)
    def _flush():
        for co in range(_C):
            stats_ref[0, 0, co] = jnp.sum(sacc_ref[co], axis=0)
            stats_ref[0, 1, co] = jnp.sum(qacc_ref[co], axis=0)


def _affine_kernel(sc_ref, sh_ref, y_ref, o_ref):
    for co in range(_C):
        o_ref[:, co, :] = (y_ref[co].astype(jnp.float32) * sc_ref[co]
                           + sh_ref[co])


def _largest_tile(n, cap):
    best = 1
    for t in range(1, min(n, cap) + 1):
        if n % t == 0:
            best = t
    return best


@jax.jit
def _forward(x, w, gamma, beta):
    n, c_in, length = x.shape
    assert c_in == _C and w.shape == (_C, _C, _K)

    tile_n = _largest_tile(n, 256)
    n_tiles = n // tile_n
    n_split = 2 if n_tiles % 2 == 0 else 1
    inner = n_tiles // n_split

    w_flat = w.astype(jnp.float32).reshape(-1)
    smem = pl.BlockSpec(memory_space=pltpu.MemorySpace.SMEM)

    y, stats = pl.pallas_call(
        _conv_stats_kernel,
        out_shape=(
            jax.ShapeDtypeStruct((_C, n, length), jnp.bfloat16),
            jax.ShapeDtypeStruct((n_split, 2, _C, length), jnp.float32),
        ),
        grid=(n_split, inner),
        in_specs=[
            smem,
            pl.BlockSpec((tile_n, _C, length),
                         lambda c, i: (c * inner + i, 0, 0)),
        ],
        out_specs=(
            pl.BlockSpec((_C, tile_n, length),
                         lambda c, i: (0, c * inner + i, 0)),
            pl.BlockSpec((1, 2, _C, length), lambda c, i: (c, 0, 0, 0)),
        ),
        scratch_shapes=[
            pltpu.VMEM((_C, tile_n, length), jnp.float32),
            pltpu.VMEM((_C, tile_n, length), jnp.float32),
        ],
        compiler_params=pltpu.CompilerParams(
            dimension_semantics=("parallel", "arbitrary"),
            vmem_limit_bytes=_VMEM_BYTES),
    )(w_flat, x)

    count = jnp.float32(n * length)
    ch_sum = jnp.sum(stats[:, 0], axis=(0, 2))
    ch_sumsq = jnp.sum(stats[:, 1], axis=(0, 2))
    mean = ch_sum / count
    var = jnp.maximum(ch_sumsq / count - mean * mean, 0.0)
    inv = jax.lax.rsqrt(var + _EPS)
    scale = gamma.astype(jnp.float32) * inv
    shift = beta.astype(jnp.float32) - mean * scale

    tile2 = _largest_tile(n, 256)
    out = pl.pallas_call(
        _affine_kernel,
        out_shape=jax.ShapeDtypeStruct((n, _C, length), x.dtype),
        grid=(n // tile2,),
        in_specs=[
            smem,
            smem,
            pl.BlockSpec((_C, tile2, length), lambda i: (0, i, 0)),
        ],
        out_specs=pl.BlockSpec((tile2, _C, length), lambda i: (i, 0, 0)),
        compiler_params=pltpu.CompilerParams(
            dimension_semantics=("parallel",),
            vmem_limit_bytes=_VMEM_BYTES),
    )(scale, shift, y)
    return out


def kernel(x, w, gamma, beta):
    return _forward(x, w, gamma, beta)
